# Initial kernel scaffold; baseline (speedup 1.0000x reference)
import jax
import jax.numpy as jnp
from jax.experimental import pallas as pl


def kernel(x, edge_index, W_in, b_in, W1, b1, W2, b2, W3, b3, W4, b4, W5, b5, W6, b6, W_out, b_out):
    raise NotImplementedError("write your pallas kernel here")


# speedup vs baseline: 10.7594x; 10.7594x over previous
import functools

import jax
import jax.numpy as jnp
from jax import lax
from jax.experimental import pallas as pl
from jax.experimental.pallas import tpu as pltpu
from jax.experimental.pallas import tpu_sc as plsc

_N = 50000
_H = 16
_E = 800000
_NW = 32
_CH = 128
_CPW = 196
_EW = _CH * _CPW
_EPAD = _EW * _NW
_NR = 50048
_SPW = _NR // 16
_BN = 2000
_GRID = _N // _BN


def _make_sc_pass(first):
    mesh = plsc.VectorSubcoreMesh(core_axis_name="c", subcore_axis_name="s")
    out_type = jax.ShapeDtypeStruct((2, _NR, _H), jnp.float32)
    scratch = [
        pltpu.VMEM((_CH,), jnp.int32),
        pltpu.VMEM((_CH,), jnp.int32),
        pltpu.VMEM((_CH, _H), jnp.float32),
        pltpu.VMEM_SHARED((_NR, _H), jnp.float32),
        pltpu.SemaphoreType.DMA,
    ]
    if first:
        scratch = scratch[:3] + [
            pltpu.VMEM((_CH, _H), jnp.float32),
        ] + scratch[3:]

    def body_fn(table, srcp, dst2, zrows, *rest):
        if first:
            ev, out, src_v, dst_v, rows_v, ev_v, acc, sem = rest
        else:
            out, src_v, dst_v, rows_v, acc, sem = rest
        c = lax.axis_index("c")
        s = lax.axis_index("s")
        wid = s * 2 + c
        pltpu.sync_copy(zrows, acc.at[pl.ds(s * _SPW, _SPW), :])
        plsc.subcore_barrier()
        base = wid * _EW
        wrow = wid * _CPW

        def body(j, carry):
            pltpu.sync_copy(srcp.at[pl.ds(base + j * _CH, _CH)], src_v)
            pltpu.sync_copy(dst2.at[wrow + j], dst_v)
            pltpu.async_copy(table.at[src_v], rows_v, sem).wait()
            pltpu.sync_copy(rows_v, acc.at[dst_v], add=True)
            if first:
                pltpu.sync_copy(ev.at[pl.ds(base + j * _CH, _CH), :], ev_v)
                pltpu.sync_copy(ev_v, acc.at[dst_v], add=True)
            return carry

        lax.fori_loop(0, _CPW, body, 0)
        plsc.subcore_barrier()
        pltpu.sync_copy(acc.at[pl.ds(s * _SPW, _SPW), :],
                        out.at[c, pl.ds(s * _SPW, _SPW), :])

    return pl.kernel(
        body_fn,
        mesh=mesh,
        out_type=out_type,
        scratch_types=scratch,
        compiler_params=pltpu.CompilerParams(use_tc_tiling_on_sc=False),
    )


_sc_pass_first = _make_sc_pass(True)
_sc_pass = _make_sc_pass(False)


def _l1_body(P_ref, x_ref, wd_ref, wb_ref, wc_ref, b_ref,
             x2_ref, m_ref, ic_ref, gm_ref):
    P = P_ref[0] + P_ref[1]
    s1 = P[:, 0:1]
    cnt = P[:, 1:2]
    g = P[:, 2:3]
    ic = 1.0 / jnp.maximum(cnt, 1.0)
    m = (cnt > 0.0).astype(jnp.float32)
    gm = g * ic
    xb = x_ref[...]
    x2_ref[...] = (m * (xb * wd_ref[...] + b_ref[...])
                   + (ic * s1) * wb_ref[...] + gm * wc_ref[...])
    m_ref[...] = m
    ic_ref[...] = ic
    gm_ref[...] = gm


def _tc_layer1(P, x, W_in, b_in):
    wd = W_in[0:1] - W_in[1:2]
    wb = W_in[1:2]
    wc = W_in[2:3]
    b = b_in.reshape(1, _H)
    vspec = pl.BlockSpec((_BN, 1), lambda i: (i, 0))
    wspec = pl.BlockSpec((1, _H), lambda i: (0, 0))
    return pl.pallas_call(
        _l1_body,
        grid=(_GRID,),
        in_specs=[
            pl.BlockSpec((2, _BN, _H), lambda i: (0, i, 0)),
            vspec, wspec, wspec, wspec, wspec,
        ],
        out_specs=[
            pl.BlockSpec((_BN, _H), lambda i: (i, 0)),
            vspec, vspec, vspec,
        ],
        out_shape=[
            jax.ShapeDtypeStruct((_N, _H), jnp.float32),
            jax.ShapeDtypeStruct((_N, 1), jnp.float32),
            jax.ShapeDtypeStruct((_N, 1), jnp.float32),
            jax.ShapeDtypeStruct((_N, 1), jnp.float32),
        ],
    )(P, x, wd, wb, wc, b)


def _make_layer_body(act, has_res):
    def body(h_ref, P_ref, m_ref, ic_ref, gm_ref, *rest):
        if has_res:
            res_ref = rest[0]
            rest = rest[1:]
        wd_ref, wb_ref, wc_ref, b_ref, o_ref = rest
        P = P_ref[0] + P_ref[1]
        o = (m_ref[...] * (jnp.dot(h_ref[...], wd_ref[...],
                                   preferred_element_type=jnp.float32)
                           + b_ref[...])
             + jnp.dot(ic_ref[...] * P, wb_ref[...],
                       preferred_element_type=jnp.float32)
             + gm_ref[...] * wc_ref[...])
        if has_res:
            o = o + res_ref[...]
        if act:
            o = jnp.where(o >= 0, o, 0.01 * o)
        o_ref[...] = o
    return body


def _tc_layer(h, P, m, ic, gm, W, b, res, act):
    wd = W[:_H] - W[_H:2 * _H]
    wb = W[_H:2 * _H]
    wc = W[2 * _H:2 * _H + 1]
    ho = W.shape[1]
    bb = b.reshape(1, ho)
    vspec = pl.BlockSpec((_BN, 1), lambda i: (i, 0))
    wspec = pl.BlockSpec((_H, ho), lambda i: (0, 0))
    sspec = pl.BlockSpec((1, ho), lambda i: (0, 0))
    ins = [h, P, m, ic, gm]
    specs = [
        pl.BlockSpec((_BN, _H), lambda i: (i, 0)),
        pl.BlockSpec((2, _BN, _H), lambda i: (0, i, 0)),
        vspec, vspec, vspec,
    ]
    if res is not None:
        ins.append(res)
        rc = res.shape[1]
        specs.append(pl.BlockSpec((_BN, rc), lambda i: (i, 0)))
    ins += [wd, wb, wc, bb]
    specs += [wspec, wspec, sspec, sspec]
    return pl.pallas_call(
        _make_layer_body(act, res is not None),
        grid=(_GRID,),
        in_specs=specs,
        out_specs=pl.BlockSpec((_BN, ho), lambda i: (i, 0)),
        out_shape=jax.ShapeDtypeStruct((_N, ho), jnp.float32),
    )(*ins)


def kernel(x, edge_index, W_in, b_in, W1, b1, W2, b2, W3, b3, W4, b4,
           W5, b5, W6, b6, W_out, b_out):
    src = edge_index[0].astype(jnp.int32)
    dst = edge_index[1].astype(jnp.int32)
    pad = _EPAD - _E
    srcp = jnp.concatenate([src, jnp.zeros((pad,), jnp.int32)])
    dstp = jnp.concatenate([dst, jnp.full((pad,), _N, jnp.int32)])
    dst2 = dstp.reshape(_EPAD // _CH, _CH)
    zrows = jnp.zeros((_SPW, _H), jnp.float32)
    ev = jnp.pad(jnp.sign((src - dst).astype(jnp.float32))[:, None],
                 ((0, pad), (2, _H - 3)))
    T0 = jnp.concatenate(
        [x, jnp.ones((_N, 1), jnp.float32),
         jnp.zeros((_N, _H - 2), jnp.float32)], axis=1)

    P1 = _sc_pass_first(T0, srcp, dst2, zrows, ev)
    x2, m, ic, gm = _tc_layer1(P1, x, W_in, b_in)

    Ws = [(W1, b1), (W2, b2), (W3, b3), (W4, b4), (W5, b5), (W6, b6)]
    for i in range(0, 6, 2):
        P = _sc_pass(x2, srcp, dst2, zrows)
        x1 = _tc_layer(x2, P, m, ic, gm, *Ws[i], res=None, act=True)
        P = _sc_pass(x1, srcp, dst2, zrows)
        x2 = _tc_layer(x1, P, m, ic, gm, *Ws[i + 1], res=x2, act=True)

    P = _sc_pass(x2, srcp, dst2, zrows)
    Wop = jnp.pad(W_out, ((0, 0), (0, _H - 1)))
    bop = jnp.pad(b_out, ((0, _H - 1),))
    y = _tc_layer(x2, P, m, ic, gm, Wop, bop, res=x, act=False)
    return y[:, 0:1]


# speedup vs baseline: 23.1941x; 2.1557x over previous
import functools

import jax
import jax.numpy as jnp
from jax import lax
from jax.experimental import pallas as pl
from jax.experimental.pallas import tpu as pltpu
from jax.experimental.pallas import tpu_sc as plsc

_N = 50000
_H = 16
_E = 800000
_NW = 32
_EW = 25088
_EPAD = _EW * _NW
_MC = 1792
_MC1 = 896
_NR = 50048
_SPW = _NR // 16
_BN = 2000
_GRID = _N // _BN


def _make_sc_pass(first):
    mesh = plsc.VectorSubcoreMesh(core_axis_name="c", subcore_axis_name="s")
    out_type = jax.ShapeDtypeStruct((2, _NR, _H), jnp.float32)
    mc = _MC1 if first else _MC
    nmc = _EW // mc
    scratch = [
        pltpu.VMEM((2, mc), jnp.int32),
        pltpu.VMEM((2, mc), jnp.int32),
        pltpu.VMEM((2, mc, _H), jnp.float32),
        pltpu.VMEM_SHARED((_NR, _H), jnp.float32),
        pltpu.SemaphoreType.DMA((2,)),
    ]
    if first:
        scratch = scratch[:3] + [
            pltpu.VMEM((2, mc, _H), jnp.float32),
            pltpu.SemaphoreType.DMA((2,)),
        ] + scratch[3:]

    def body_fn(table, srcp, dstp, zrows, *rest):
        if first:
            ev, out, src_v, dst_v, rows_v, ev_v, sem_e, acc, sem = rest
        else:
            out, src_v, dst_v, rows_v, acc, sem = rest
        c = lax.axis_index("c")
        s = lax.axis_index("s")
        wid = s * 2 + c
        pltpu.sync_copy(zrows, acc.at[pl.ds(s * _SPW, _SPW), :])
        plsc.subcore_barrier()
        base = wid * _EW

        def fetch(j, p):
            pltpu.sync_copy(srcp.at[pl.ds(base + j * mc, mc)], src_v.at[p])
            pltpu.sync_copy(dstp.at[pl.ds(base + j * mc, mc)], dst_v.at[p])
            pltpu.async_copy(table.at[src_v.at[p]], rows_v.at[p], sem.at[p])
            if first:
                pltpu.async_copy(ev.at[pl.ds(base + j * mc, mc), :],
                                 ev_v.at[p], sem_e.at[p])

        fetch(0, 0)

        def body(t, carry):
            for p in (0, 1):
                tc = 2 * t + p

                @pl.when(tc + 1 < nmc)
                def _():
                    fetch(tc + 1, 1 - p)

                pltpu.make_async_copy(
                    table.at[src_v.at[p]], rows_v.at[p], sem.at[p]).wait()
                pltpu.sync_copy(rows_v.at[p], acc.at[dst_v.at[p]], add=True)
                if first:
                    pltpu.make_async_copy(
                        ev.at[pl.ds(base, mc), :], ev_v.at[p],
                        sem_e.at[p]).wait()
                    pltpu.sync_copy(ev_v.at[p], acc.at[dst_v.at[p]], add=True)
            return carry

        lax.fori_loop(0, nmc // 2, body, 0)
        plsc.subcore_barrier()
        pltpu.sync_copy(acc.at[pl.ds(s * _SPW, _SPW), :],
                        out.at[c, pl.ds(s * _SPW, _SPW), :])

    return pl.kernel(
        body_fn,
        mesh=mesh,
        out_type=out_type,
        scratch_types=scratch,
        compiler_params=pltpu.CompilerParams(use_tc_tiling_on_sc=False),
    )


_sc_pass_first = _make_sc_pass(True)
_sc_pass = _make_sc_pass(False)


def _l1_body(P_ref, x_ref, wd_ref, wb_ref, wc_ref, b_ref,
             x2_ref, m_ref, ic_ref, gm_ref):
    P = P_ref[0] + P_ref[1]
    s1 = P[:, 0:1]
    cnt = P[:, 1:2]
    g = P[:, 2:3]
    ic = 1.0 / jnp.maximum(cnt, 1.0)
    m = (cnt > 0.0).astype(jnp.float32)
    gm = g * ic
    xb = x_ref[...]
    x2_ref[...] = (m * (xb * wd_ref[...] + b_ref[...])
                   + (ic * s1) * wb_ref[...] + gm * wc_ref[...])
    m_ref[...] = m
    ic_ref[...] = ic
    gm_ref[...] = gm


def _tc_layer1(P, x, W_in, b_in):
    wd = W_in[0:1] - W_in[1:2]
    wb = W_in[1:2]
    wc = W_in[2:3]
    b = b_in.reshape(1, _H)
    vspec = pl.BlockSpec((_BN, 1), lambda i: (i, 0))
    wspec = pl.BlockSpec((1, _H), lambda i: (0, 0))
    return pl.pallas_call(
        _l1_body,
        grid=(_GRID,),
        in_specs=[
            pl.BlockSpec((2, _BN, _H), lambda i: (0, i, 0)),
            vspec, wspec, wspec, wspec, wspec,
        ],
        out_specs=[
            pl.BlockSpec((_BN, _H), lambda i: (i, 0)),
            vspec, vspec, vspec,
        ],
        out_shape=[
            jax.ShapeDtypeStruct((_N, _H), jnp.float32),
            jax.ShapeDtypeStruct((_N, 1), jnp.float32),
            jax.ShapeDtypeStruct((_N, 1), jnp.float32),
            jax.ShapeDtypeStruct((_N, 1), jnp.float32),
        ],
    )(P, x, wd, wb, wc, b)


def _make_layer_body(act, has_res):
    def body(h_ref, P_ref, m_ref, ic_ref, gm_ref, *rest):
        if has_res:
            res_ref = rest[0]
            rest = rest[1:]
        wd_ref, wb_ref, wc_ref, b_ref, o_ref = rest
        P = P_ref[0] + P_ref[1]
        o = (m_ref[...] * (jnp.dot(h_ref[...], wd_ref[...],
                                   preferred_element_type=jnp.float32)
                           + b_ref[...])
             + jnp.dot(ic_ref[...] * P, wb_ref[...],
                       preferred_element_type=jnp.float32)
             + gm_ref[...] * wc_ref[...])
        if has_res:
            o = o + res_ref[...]
        if act:
            o = jnp.where(o >= 0, o, 0.01 * o)
        o_ref[...] = o
    return body


def _tc_layer(h, P, m, ic, gm, W, b, res, act):
    wd = W[:_H] - W[_H:2 * _H]
    wb = W[_H:2 * _H]
    wc = W[2 * _H:2 * _H + 1]
    ho = W.shape[1]
    bb = b.reshape(1, ho)
    vspec = pl.BlockSpec((_BN, 1), lambda i: (i, 0))
    wspec = pl.BlockSpec((_H, ho), lambda i: (0, 0))
    sspec = pl.BlockSpec((1, ho), lambda i: (0, 0))
    ins = [h, P, m, ic, gm]
    specs = [
        pl.BlockSpec((_BN, _H), lambda i: (i, 0)),
        pl.BlockSpec((2, _BN, _H), lambda i: (0, i, 0)),
        vspec, vspec, vspec,
    ]
    if res is not None:
        ins.append(res)
        rc = res.shape[1]
        specs.append(pl.BlockSpec((_BN, rc), lambda i: (i, 0)))
    ins += [wd, wb, wc, bb]
    specs += [wspec, wspec, sspec, sspec]
    return pl.pallas_call(
        _make_layer_body(act, res is not None),
        grid=(_GRID,),
        in_specs=specs,
        out_specs=pl.BlockSpec((_BN, ho), lambda i: (i, 0)),
        out_shape=jax.ShapeDtypeStruct((_N, ho), jnp.float32),
    )(*ins)


def kernel(x, edge_index, W_in, b_in, W1, b1, W2, b2, W3, b3, W4, b4,
           W5, b5, W6, b6, W_out, b_out):
    src = edge_index[0].astype(jnp.int32)
    dst = edge_index[1].astype(jnp.int32)
    pad = _EPAD - _E
    srcp = jnp.concatenate([src, jnp.zeros((pad,), jnp.int32)])
    dstp = jnp.concatenate([dst, jnp.full((pad,), _N, jnp.int32)])
    zrows = jnp.zeros((_SPW, _H), jnp.float32)
    ev = jnp.pad(jnp.sign((src - dst).astype(jnp.float32))[:, None],
                 ((0, pad), (2, _H - 3)))
    T0 = jnp.concatenate(
        [x, jnp.ones((_N, 1), jnp.float32),
         jnp.zeros((_N, _H - 2), jnp.float32)], axis=1)

    P1 = _sc_pass_first(T0, srcp, dstp, zrows, ev)
    x2, m, ic, gm = _tc_layer1(P1, x, W_in, b_in)

    Ws = [(W1, b1), (W2, b2), (W3, b3), (W4, b4), (W5, b5), (W6, b6)]
    for i in range(0, 6, 2):
        P = _sc_pass(x2, srcp, dstp, zrows)
        x1 = _tc_layer(x2, P, m, ic, gm, *Ws[i], res=None, act=True)
        P = _sc_pass(x1, srcp, dstp, zrows)
        x2 = _tc_layer(x1, P, m, ic, gm, *Ws[i + 1], res=x2, act=True)

    P = _sc_pass(x2, srcp, dstp, zrows)
    Wop = jnp.pad(W_out, ((0, 0), (0, _H - 1)))
    bop = jnp.pad(b_out, ((0, _H - 1),))
    y = _tc_layer(x2, P, m, ic, gm, Wop, bop, res=x, act=False)
    return y[:, 0:1]
